# baseline (speedup 1.0000x reference)
import functools

import jax
import jax.numpy as jnp
from jax import lax
from jax.experimental import pallas as pl
from jax.experimental.pallas import tpu as pltpu
from jax.experimental.pallas import tpu_sc as plsc

_INFO = plsc.get_sparse_core_info()
_NC = _INFO.num_cores
_NS = _INFO.num_subcores
_NW = _NC * _NS


def _project_table(emb, W, b2d):
    V, D = emb.shape[0], W.shape[1]

    def body(emb_ref, w_ref, b_ref, out_ref):
        out_ref[...] = (
            jnp.dot(emb_ref[...], w_ref[...], preferred_element_type=jnp.float32)
            + b_ref[...]
        )

    return pl.pallas_call(
        body,
        out_shape=jax.ShapeDtypeStruct((V, D), jnp.float32),
    )(emb, W, b2d)


def _make_gather(B, D, chunk):
    n_ch = B // (_NW * chunk)
    b_per_w = B // _NW
    mesh = plsc.VectorSubcoreMesh(core_axis_name="c", subcore_axis_name="s")

    @functools.partial(
        pl.kernel,
        mesh=mesh,
        out_type=jax.ShapeDtypeStruct((B, D), jnp.float32),
        scratch_types=[
            pltpu.VMEM((chunk,), jnp.int32),
            pltpu.VMEM((chunk, D), jnp.float32),
            pltpu.SemaphoreType.DMA,
        ],
    )
    def gather_kernel(table_hbm, idx_hbm, out_hbm, idx_v, rows_v, sem):
        wid = lax.axis_index("s") * _NC + lax.axis_index("c")
        base = wid * b_per_w

        def body(i, carry):
            off = base + i * chunk
            pltpu.sync_copy(idx_hbm.at[pl.ds(off, chunk)], idx_v)
            pltpu.async_copy(table_hbm.at[idx_v], rows_v, sem).wait()
            pltpu.sync_copy(rows_v, out_hbm.at[pl.ds(off, chunk)])
            return carry

        lax.fori_loop(0, n_ch, body, 0)

    return gather_kernel


def kernel(actions, embedding_table, W, b):
    Bb, L = actions.shape
    D = W.shape[1]
    B = Bb * L

    table = _project_table(embedding_table, W, b.reshape(1, D))
    idx = actions.reshape(B)

    chunk = 128
    out = _make_gather(B, D, chunk)(table, idx)
    return out.reshape(Bb, L, D)


# speedup vs baseline: 1.0218x; 1.0218x over previous
import functools

import jax
import jax.numpy as jnp
from jax import lax
from jax.experimental import pallas as pl
from jax.experimental.pallas import tpu as pltpu
from jax.experimental.pallas import tpu_sc as plsc

_INFO = plsc.get_sparse_core_info()
_NC = _INFO.num_cores
_NS = _INFO.num_subcores
_NW = _NC * _NS


def _project_table(emb, W, b2d):
    V, D = emb.shape[0], W.shape[1]

    def body(emb_ref, w_ref, b_ref, out_ref):
        out_ref[...] = (
            jnp.dot(emb_ref[...], w_ref[...], preferred_element_type=jnp.float32)
            + b_ref[...]
        )

    return pl.pallas_call(
        body,
        out_shape=jax.ShapeDtypeStruct((V, D), jnp.float32),
    )(emb, W, b2d)


def _make_gather(B, D, chunk):
    b_per_w = B // _NW
    n_ch = b_per_w // chunk
    n_pairs = n_ch // 2
    mesh = plsc.VectorSubcoreMesh(core_axis_name="c", subcore_axis_name="s")

    @functools.partial(
        pl.kernel,
        mesh=mesh,
        out_type=jax.ShapeDtypeStruct((B, D), jnp.float32),
        scratch_types=[
            pltpu.VMEM((b_per_w,), jnp.int32),
            pltpu.VMEM((chunk, D), jnp.float32),
            pltpu.VMEM((chunk, D), jnp.float32),
            pltpu.SemaphoreType.DMA,
            pltpu.SemaphoreType.DMA,
            pltpu.SemaphoreType.DMA,
            pltpu.SemaphoreType.DMA,
        ],
    )
    def gather_kernel(table_hbm, idx_hbm, out_hbm, idx_all, rows0, rows1,
                      gsem0, gsem1, wsem0, wsem1):
        wid = lax.axis_index("s") * _NC + lax.axis_index("c")
        base = wid * b_per_w

        rows = (rows0, rows1)
        gsem = (gsem0, gsem1)
        wsem = (wsem0, wsem1)

        pltpu.sync_copy(idx_hbm.at[pl.ds(base, b_per_w)], idx_all)

        def gather_issue(c, bf):
            pltpu.async_copy(
                table_hbm.at[idx_all.at[pl.ds(c * chunk, chunk)]],
                rows[bf], gsem[bf])

        def gather_wait(bf):
            pltpu.make_async_copy(
                out_hbm.at[pl.ds(0, chunk)], rows[bf], gsem[bf]).wait()

        def write_issue(c, bf):
            pltpu.async_copy(
                rows[bf], out_hbm.at[pl.ds(base + c * chunk, chunk)], wsem[bf])

        def write_wait(bf):
            pltpu.make_async_copy(
                rows[bf], out_hbm.at[pl.ds(0, chunk)], wsem[bf]).wait()

        gather_issue(0, 0)
        gather_issue(1, 1)

        def body(p, carry):
            c0 = 2 * p
            gather_wait(0)
            write_issue(c0, 0)
            gather_wait(1)
            write_issue(c0 + 1, 1)

            @pl.when(p < n_pairs - 1)
            def _prefetch():
                write_wait(0)
                gather_issue(c0 + 2, 0)
                write_wait(1)
                gather_issue(c0 + 3, 1)

            return carry

        lax.fori_loop(0, n_pairs, body, 0)
        write_wait(0)
        write_wait(1)

    return gather_kernel


def kernel(actions, embedding_table, W, b):
    Bb, L = actions.shape
    D = W.shape[1]
    B = Bb * L

    table = _project_table(embedding_table, W, b.reshape(1, D))
    idx = actions.reshape(B)

    chunk = 80
    out = _make_gather(B, D, chunk)(table, idx)
    return out.reshape(Bb, L, D)


# speedup vs baseline: 1.0556x; 1.0331x over previous
import functools

import jax
import jax.numpy as jnp
from jax import lax
from jax.experimental import pallas as pl
from jax.experimental.pallas import tpu as pltpu
from jax.experimental.pallas import tpu_sc as plsc

_INFO = plsc.get_sparse_core_info()
_NC = _INFO.num_cores
_NS = _INFO.num_subcores
_NW = _NC * _NS
_REP = 8


def _project_table(emb, W, b2d):
    V, D = emb.shape[0], W.shape[1]

    def body(emb_ref, w_ref, b_ref, out_ref):
        out_ref[...] = (
            jnp.dot(emb_ref[...], w_ref[...], preferred_element_type=jnp.float32)
            + b_ref[...]
        )

    return pl.pallas_call(
        body,
        grid=(_REP,),
        in_specs=[
            pl.BlockSpec((V, emb.shape[1]), lambda r: (0, 0)),
            pl.BlockSpec((emb.shape[1], D), lambda r: (0, 0)),
            pl.BlockSpec((1, D), lambda r: (0, 0)),
        ],
        out_specs=pl.BlockSpec((V, D), lambda r: (r, 0)),
        out_shape=jax.ShapeDtypeStruct((_REP * V, D), jnp.float32),
    )(emb, W, b2d)


def _make_gather(B, D, V, chunk):
    b_per_w = B // _NW
    n_ch = b_per_w // chunk
    n_pairs = n_ch // 2
    mesh = plsc.VectorSubcoreMesh(core_axis_name="c", subcore_axis_name="s")

    @functools.partial(
        pl.kernel,
        mesh=mesh,
        out_type=jax.ShapeDtypeStruct((B, D), jnp.float32),
        scratch_types=[
            pltpu.VMEM((b_per_w,), jnp.int32),
            pltpu.VMEM((chunk, D), jnp.float32),
            pltpu.VMEM((chunk, D), jnp.float32),
            pltpu.SemaphoreType.DMA,
            pltpu.SemaphoreType.DMA,
            pltpu.SemaphoreType.DMA,
            pltpu.SemaphoreType.DMA,
        ],
    )
    def gather_kernel(table_hbm, idx_hbm, out_hbm, idx_all,
                      rows0, rows1, gsem0, gsem1, wsem0, wsem1):
        wid = lax.axis_index("s") * _NC + lax.axis_index("c")
        base = wid * b_per_w

        rows = (rows0, rows1)
        gsem = (gsem0, gsem1)
        wsem = (wsem0, wsem1)

        pltpu.sync_copy(idx_hbm.at[pl.ds(base, b_per_w)], idx_all)
        rep_off = lax.rem(wid, _REP) * V

        def reloc(k, carry):
            sl = pl.ds(k * 16, 16)
            idx_all[sl] = idx_all[sl] + rep_off
            return carry

        lax.fori_loop(0, b_per_w // 16, reloc, 0)

        def gather_issue(c, bf):
            pltpu.async_copy(
                table_hbm.at[idx_all.at[pl.ds(c * chunk, chunk)]],
                rows[bf], gsem[bf])

        def gather_wait(bf):
            pltpu.make_async_copy(
                out_hbm.at[pl.ds(0, chunk)], rows[bf], gsem[bf]).wait()

        def write_issue(c, bf):
            pltpu.async_copy(
                rows[bf], out_hbm.at[pl.ds(base + c * chunk, chunk)], wsem[bf])

        def write_wait(bf):
            pltpu.make_async_copy(
                rows[bf], out_hbm.at[pl.ds(0, chunk)], wsem[bf]).wait()

        gather_issue(0, 0)
        gather_issue(1, 1)

        def body(p, carry):
            c0 = 2 * p
            gather_wait(0)
            write_issue(c0, 0)
            gather_wait(1)
            write_issue(c0 + 1, 1)

            @pl.when(p < n_pairs - 1)
            def _prefetch():
                write_wait(0)
                gather_issue(c0 + 2, 0)
                write_wait(1)
                gather_issue(c0 + 3, 1)

            return carry

        lax.fori_loop(0, n_pairs, body, 0)
        write_wait(0)
        write_wait(1)

    return gather_kernel


def kernel(actions, embedding_table, W, b):
    Bb, L = actions.shape
    D = W.shape[1]
    B = Bb * L
    V = embedding_table.shape[0]

    table = _project_table(embedding_table, W, b.reshape(1, D))
    idx = actions.reshape(B)

    chunk = 80
    out = _make_gather(B, D, V, chunk)(table, idx)
    return out.reshape(Bb, L, D)


# speedup vs baseline: 3.5380x; 3.3516x over previous
import functools

import jax
import jax.numpy as jnp
from jax import lax
from jax.experimental import pallas as pl
from jax.experimental.pallas import tpu as pltpu
from jax.experimental.pallas import tpu_sc as plsc

_INFO = plsc.get_sparse_core_info()
_NC = _INFO.num_cores
_NS = _INFO.num_subcores
_NW = _NC * _NS
_REP = 8


def _project_table(emb, W, b2d):
    V, D = emb.shape[0], W.shape[1]

    def body(emb_ref, w_ref, b_ref, out_ref):
        out_ref[...] = (
            jnp.dot(emb_ref[...], w_ref[...], preferred_element_type=jnp.float32)
            + b_ref[...]
        )

    return pl.pallas_call(
        body,
        grid=(_REP,),
        in_specs=[
            pl.BlockSpec((V, emb.shape[1]), lambda r: (0, 0)),
            pl.BlockSpec((emb.shape[1], D), lambda r: (0, 0)),
            pl.BlockSpec((1, D), lambda r: (0, 0)),
        ],
        out_specs=pl.BlockSpec((V, D), lambda r: (r, 0)),
        out_shape=jax.ShapeDtypeStruct((_REP * V, D), jnp.float32),
    )(emb, W, b2d)


def _make_gather(B, D, V, chunk):
    b_per_w = B // _NW
    n_ch = b_per_w // chunk
    n_pairs = n_ch // 2
    mesh = plsc.VectorSubcoreMesh(core_axis_name="c", subcore_axis_name="s")

    @functools.partial(
        pl.kernel,
        mesh=mesh,
        out_type=jax.ShapeDtypeStruct((B, D), jnp.float32),
        scratch_types=[
            pltpu.VMEM((b_per_w,), jnp.int32),
            pltpu.VMEM((chunk, D), jnp.float32),
            pltpu.VMEM((chunk, D), jnp.float32),
            pltpu.SemaphoreType.DMA,
            pltpu.SemaphoreType.DMA,
            pltpu.SemaphoreType.DMA,
            pltpu.SemaphoreType.DMA,
        ],
    )
    def gather_kernel(table_hbm, idx_hbm, out_hbm, idx_all,
                      rows0, rows1, gsem0, gsem1, wsem0, wsem1):
        wid = lax.axis_index("s") * _NC + lax.axis_index("c")
        base = wid * b_per_w

        rows = (rows0, rows1)
        gsem = (gsem0, gsem1)
        wsem = (wsem0, wsem1)

        pltpu.sync_copy(idx_hbm.at[pl.ds(base, b_per_w)], idx_all)
        rep_off = lax.rem(wid, _REP) * V

        def reloc(k, carry):
            sl = pl.ds(k * 16, 16)
            idx_all[sl] = idx_all[sl] + rep_off
            return carry

        lax.fori_loop(0, b_per_w // 16, reloc, 0)

        def gather_issue(c, bf):
            pltpu.async_copy(
                table_hbm.at[idx_all.at[pl.ds(c * chunk, chunk)]],
                rows[bf], gsem[bf])

        def gather_wait(bf):
            pltpu.make_async_copy(
                out_hbm.at[pl.ds(0, chunk)], rows[bf], gsem[bf]).wait()

        def write_issue(c, bf):
            pltpu.async_copy(
                rows[bf], out_hbm.at[pl.ds(base + c * chunk, chunk)], wsem[bf])

        def write_wait(bf):
            pltpu.make_async_copy(
                rows[bf], out_hbm.at[pl.ds(0, chunk)], wsem[bf]).wait()

        gather_issue(0, 0)
        gather_issue(1, 1)

        def body(p, carry):
            c0 = 2 * p
            gather_wait(0)
            write_issue(c0, 0)
            gather_wait(1)
            write_issue(c0 + 1, 1)

            @pl.when(p < n_pairs - 1)
            def _prefetch():
                write_wait(0)
                gather_issue(c0 + 2, 0)
                write_wait(1)
                gather_issue(c0 + 3, 1)

            return carry

        lax.fori_loop(0, n_pairs, body, 0)
        write_wait(0)
        write_wait(1)

    return gather_kernel


def kernel(actions, embedding_table, W, b):
    Bb, L = actions.shape
    D = W.shape[1]
    B = Bb * L
    V = embedding_table.shape[0]

    table = _project_table(embedding_table, W, b.reshape(1, D))
    idx = actions.T.reshape(B)

    chunk = 80
    out = _make_gather(B, D, V, chunk)(table, idx)
    return out.reshape(L, Bb, D).transpose(1, 0, 2)
